# baseline (speedup 1.0000x reference)
import functools

import jax
import jax.numpy as jnp
from jax import lax
from jax.experimental import pallas as pl
from jax.experimental.pallas import tpu as pltpu
from jax.experimental.pallas import tpu_sc as plsc

D = 64
CHUNK = 128
NTILES = 16
NCORES = 2
HALF = 25000
ACC_ROWS = 196 * 128
OUT_PER_TILE = 1560
NBUF = 3


def _spmm_body(nchunk, pk_hbm, table_hbm, out_hbm,
               pkb4, rows4, sidx4, acc_sh, *sems):
    c = lax.axis_index("c")
    s = lax.axis_index("s")
    si = sems[0:NBUF]
    sg = sems[NBUF:2 * NBUF]
    ss = sems[2 * NBUF:3 * NBUF]

    zero16 = jnp.zeros((16,), jnp.float32)

    def zrow(e, _):
        for j in range(D // 16):
            rows4[0, e, pl.ds(16 * j, 16)] = zero16
        return _

    lax.fori_loop(0, CHUNK, zrow, 0)
    nzc = ACC_ROWS // CHUNK
    for k in range(-(-nzc // NTILES)):
        zc = k * NTILES + s

        @pl.when(zc < nzc)
        def _():
            pltpu.sync_copy(rows4.at[0], acc_sh.at[pl.ds(zc * CHUNK, CHUNK)])
    plsc.subcore_barrier()

    base_cid = s * nchunk

    def start_idx(ci, b):
        pltpu.async_copy(pk_hbm.at[base_cid + ci], pkb4.at[b], si[b])

    def wait_idx(ci, b):
        pltpu.make_async_copy(pk_hbm.at[base_cid + ci], pkb4.at[b],
                              si[b]).wait()

    def start_gather(b):
        pltpu.async_copy(table_hbm.at[pkb4.at[b].at[1]], rows4.at[b], sg[b])

    def wait_gather(b):
        pltpu.make_async_copy(table_hbm.at[pkb4.at[b].at[1]], rows4.at[b],
                              sg[b]).wait()

    def start_scatter(b):
        pltpu.async_copy(rows4.at[b], acc_sh.at[sidx4.at[b]], ss[b], add=True)

    def wait_scatter(b):
        pltpu.make_async_copy(rows4.at[b], acc_sh.at[sidx4.at[b]],
                              ss[b]).wait()

    def compute(b):
        half_base = c * HALF
        for j in range(CHUNK // 16):
            d16 = pkb4[b, 0, pl.ds(16 * j, 16)]
            loc = d16 - half_base
            ok = (loc >= 0) & (loc < HALF)
            sidx4[b, pl.ds(16 * j, 16)] = jnp.where(ok, loc, HALF)
        for j in range(CHUNK // 16):
            vv = plsc.bitcast(pkb4[b, 2, pl.ds(16 * j, 16)], jnp.float32)
            for t in range(16):
                e = 16 * j + t
                v = vv[t]
                for q in range(D // 16):
                    rows4[b, e, pl.ds(16 * q, 16)] = (
                        rows4[b, e, pl.ds(16 * q, 16)] * v)

    for k in range(NBUF):
        start_idx(k, k)
    for k in range(NBUF - 1):
        wait_idx(k, k)
        start_gather(k)

    def ring_body(cg, carry):
        for b in range(NBUF):
            ci = cg * NBUF + b
            b1 = (b + NBUF - 1) % NBUF

            @pl.when((ci >= 1) & (ci + NBUF - 1 < nchunk))
            def _():
                wait_scatter(b1)

            @pl.when(ci + NBUF - 1 < nchunk)
            def _():
                wait_idx(ci + NBUF - 1, b1)
                start_gather(b1)

            wait_gather(b)
            compute(b)

            @pl.when(ci + NBUF < nchunk)
            def _():
                start_idx(ci + NBUF, b)

            start_scatter(b)
        return carry

    lax.fori_loop(0, nchunk // NBUF, ring_body, 0)
    for b in range(NBUF):
        wait_scatter(b)
    plsc.subcore_barrier()

    @pl.when(s < NTILES - 1)
    def _():
        pltpu.sync_copy(
            acc_sh.at[pl.ds(s * OUT_PER_TILE, OUT_PER_TILE)],
            out_hbm.at[pl.ds(c * HALF + s * OUT_PER_TILE, OUT_PER_TILE)])

    @pl.when(s == NTILES - 1)
    def _():
        base = (NTILES - 1) * OUT_PER_TILE
        last = HALF - base
        pltpu.sync_copy(
            acc_sh.at[pl.ds(base, last)],
            out_hbm.at[pl.ds(c * HALF + base, last)])


@functools.cache
def _make_spmm(n_rows, nchunk):
    return pl.kernel(
        functools.partial(_spmm_body, nchunk),
        out_type=jax.ShapeDtypeStruct((n_rows, D), jnp.float32),
        mesh=plsc.VectorSubcoreMesh(core_axis_name="c", subcore_axis_name="s"),
        compiler_params=pltpu.CompilerParams(use_tc_tiling_on_sc=False,
                                             needs_layout_passes=False),
        scratch_types=[
            pltpu.VMEM((NBUF, 3, CHUNK), jnp.int32),
            pltpu.VMEM((NBUF, CHUNK, D), jnp.float32),
            pltpu.VMEM((NBUF, CHUNK), jnp.int32),
            pltpu.VMEM_SHARED((ACC_ROWS, D), jnp.float32),
        ] + [pltpu.SemaphoreType.DMA] * (3 * NBUF),
    )


def _sums_body(e0u, z1u, z2u, z3u, e0i, z1i, z2i, z3i,
               eu, su, ei, si):
    pu = e0u[...] + z1u[...] + z2u[...]
    su[...] = pu
    eu[...] = pu + z3u[...]
    pi = e0i[...] + z1i[...] + z2i[...]
    si[...] = pi
    ei[...] = pi + z3i[...]


def _kt_body(vtp, s_i, pu):
    pu[...] = jnp.dot(vtp[...], s_i[...], preferred_element_type=jnp.float32)


def _g_body(e0u, up, pu, e0i, vp, qi, gu, gi):
    gu[...] = e0u[...] + jnp.dot(up[...], pu[...],
                                 preferred_element_type=jnp.float32)
    gi[...] = e0i[...] + jnp.dot(vp[...], qi[...],
                                 preferred_element_type=jnp.float32)


def kernel(adj_indices, adj_values, E_u_0, E_i_0, u_mul_s, v_mul_s, ut, vt):
    n_users, _ = E_u_0.shape
    n_items, _ = E_i_0.shape
    nnz = adj_values.shape[0]
    row = adj_indices[0].astype(jnp.int32)
    col = adj_indices[1].astype(jnp.int32)
    val = adj_values.astype(jnp.float32)

    grp = NTILES * CHUNK * NBUF
    epad = -(-nnz // grp) * grp
    pad = epad - nnz
    if pad:
        row = jnp.concatenate([row, jnp.zeros((pad,), jnp.int32)])
        col = jnp.concatenate([col, jnp.zeros((pad,), jnp.int32)])
        val = jnp.concatenate([val, jnp.zeros((pad,), jnp.float32)])
    nchunk = epad // (NTILES * CHUNK)

    vbits = jax.lax.bitcast_convert_type(val, jnp.int32)
    pk_u = jnp.stack([row, col, vbits]).reshape(3, -1, CHUNK).transpose(1, 0, 2)
    pk_i = jnp.stack([col, row, vbits]).reshape(3, -1, CHUNK).transpose(1, 0, 2)

    spmm_u = _make_spmm(n_users, nchunk)
    spmm_i = _make_spmm(n_items, nchunk)

    Zu1 = spmm_u(pk_u, E_i_0)
    Zi1 = spmm_i(pk_i, E_u_0)
    Zu2 = spmm_u(pk_u, Zi1)
    Zi2 = spmm_i(pk_i, Zu1)
    Zu3 = spmm_u(pk_u, Zi2)
    Zi3 = spmm_i(pk_i, Zu2)

    nb = 50
    bu = n_users // nb
    bi = n_items // nb
    blk_u = pl.BlockSpec((bu, D), lambda i: (i, 0))
    blk_i = pl.BlockSpec((bi, D), lambda i: (i, 0))
    E_u, S_u, E_i, S_i = pl.pallas_call(
        _sums_body,
        grid=(nb,),
        in_specs=[blk_u] * 4 + [blk_i] * 4,
        out_specs=[blk_u, blk_u, blk_i, blk_i],
        out_shape=[jax.ShapeDtypeStruct((n_users, D), jnp.float32)] * 2
        + [jax.ShapeDtypeStruct((n_items, D), jnp.float32)] * 2,
    )(E_u_0, Zu1, Zu2, Zu3, E_i_0, Zi1, Zi2, Zi3)

    q = ut.shape[0]
    qp = 32
    vtp = jnp.pad(vt, ((0, qp - q), (0, 0)))
    utp = jnp.pad(ut, ((0, qp - q), (0, 0)))
    up = jnp.pad(u_mul_s, ((0, 0), (0, qp - q)))
    vp = jnp.pad(v_mul_s, ((0, 0), (0, qp - q)))

    P_u = pl.pallas_call(
        _kt_body,
        out_shape=jax.ShapeDtypeStruct((qp, D), jnp.float32),
    )(vtp, S_i)
    Q_i = pl.pallas_call(
        _kt_body,
        out_shape=jax.ShapeDtypeStruct((qp, D), jnp.float32),
    )(utp, S_u)

    G_u, G_i = pl.pallas_call(
        _g_body,
        grid=(nb,),
        in_specs=[
            blk_u,
            pl.BlockSpec((bu, qp), lambda i: (i, 0)),
            pl.BlockSpec((qp, D), lambda i: (0, 0)),
            blk_i,
            pl.BlockSpec((bi, qp), lambda i: (i, 0)),
            pl.BlockSpec((qp, D), lambda i: (0, 0)),
        ],
        out_specs=[blk_u, blk_i],
        out_shape=[jax.ShapeDtypeStruct((n_users, D), jnp.float32),
                   jax.ShapeDtypeStruct((n_items, D), jnp.float32)],
    )(E_u_0, up, P_u, E_i_0, vp, Q_i)

    return (E_u, E_i, G_u, G_i)


# speedup vs baseline: 1.0943x; 1.0943x over previous
import functools

import jax
import jax.numpy as jnp
from jax import lax
from jax.experimental import pallas as pl
from jax.experimental.pallas import tpu as pltpu
from jax.experimental.pallas import tpu_sc as plsc

D = 64
CHUNK = 128
NTILES = 16
NCORES = 2
HALF = 25000
ACC_ROWS = 196 * 128
OUT_PER_TILE = 1560
NBUF = 3


def _spmm_body(nchunk, pk_hbm, table_hbm, out_hbm,
               pkb4, rows4, sidx4, acc_sh, *sems):
    c = lax.axis_index("c")
    s = lax.axis_index("s")
    si = sems[0:NBUF]
    sg = sems[NBUF:2 * NBUF]
    ss = sems[2 * NBUF:3 * NBUF]

    zero16 = jnp.zeros((16,), jnp.float32)

    def zrow(e, _):
        for j in range(D // 16):
            rows4[0, e, pl.ds(16 * j, 16)] = zero16
        return _

    lax.fori_loop(0, CHUNK, zrow, 0)
    nzc = ACC_ROWS // CHUNK
    for k in range(-(-nzc // NTILES)):
        zc = k * NTILES + s

        @pl.when(zc < nzc)
        def _():
            pltpu.sync_copy(rows4.at[0], acc_sh.at[pl.ds(zc * CHUNK, CHUNK)])
    plsc.subcore_barrier()

    base_cid = s * nchunk

    def start_idx(ci, b):
        pltpu.async_copy(pk_hbm.at[base_cid + ci], pkb4.at[b], si[b])

    def wait_idx(ci, b):
        pltpu.make_async_copy(pk_hbm.at[base_cid + ci], pkb4.at[b],
                              si[b]).wait()

    def start_gather(b):
        pltpu.async_copy(table_hbm.at[pkb4.at[b].at[1]], rows4.at[b], sg[b])

    def wait_gather(b):
        pltpu.make_async_copy(table_hbm.at[pkb4.at[b].at[1]], rows4.at[b],
                              sg[b]).wait()

    def start_scatter(b):
        pltpu.async_copy(rows4.at[b], acc_sh.at[sidx4.at[b]], ss[b], add=True)

    def wait_scatter(b):
        pltpu.make_async_copy(rows4.at[b], acc_sh.at[sidx4.at[b]],
                              ss[b]).wait()

    def compute(b):
        half_base = c * HALF
        trash = HALF + lax.iota(jnp.int32, 16)
        for j in range(CHUNK // 16):
            d16 = pkb4[b, 0, pl.ds(16 * j, 16)]
            loc = d16 - half_base
            ok = (loc >= 0) & (loc < HALF)
            sidx4[b, pl.ds(16 * j, 16)] = jnp.where(ok, loc, trash)
        for j in range(CHUNK // 16):
            vv = plsc.bitcast(pkb4[b, 2, pl.ds(16 * j, 16)], jnp.float32)
            for t in range(16):
                e = 16 * j + t
                v = vv[t]
                for q in range(D // 16):
                    rows4[b, e, pl.ds(16 * q, 16)] = (
                        rows4[b, e, pl.ds(16 * q, 16)] * v)

    for k in range(NBUF):
        start_idx(k, k)
    for k in range(NBUF - 1):
        wait_idx(k, k)
        start_gather(k)

    def ring_body(cg, carry):
        for b in range(NBUF):
            ci = cg * NBUF + b
            b1 = (b + NBUF - 1) % NBUF

            @pl.when((ci >= 1) & (ci + NBUF - 1 < nchunk))
            def _():
                wait_scatter(b1)

            @pl.when(ci + NBUF - 1 < nchunk)
            def _():
                wait_idx(ci + NBUF - 1, b1)
                start_gather(b1)

            wait_gather(b)
            compute(b)

            @pl.when(ci + NBUF < nchunk)
            def _():
                start_idx(ci + NBUF, b)

            start_scatter(b)
        return carry

    lax.fori_loop(0, nchunk // NBUF, ring_body, 0)
    for b in range(NBUF):
        wait_scatter(b)
    plsc.subcore_barrier()

    @pl.when(s < NTILES - 1)
    def _():
        pltpu.sync_copy(
            acc_sh.at[pl.ds(s * OUT_PER_TILE, OUT_PER_TILE)],
            out_hbm.at[pl.ds(c * HALF + s * OUT_PER_TILE, OUT_PER_TILE)])

    @pl.when(s == NTILES - 1)
    def _():
        base = (NTILES - 1) * OUT_PER_TILE
        last = HALF - base
        pltpu.sync_copy(
            acc_sh.at[pl.ds(base, last)],
            out_hbm.at[pl.ds(c * HALF + base, last)])


@functools.cache
def _make_spmm(n_rows, nchunk):
    return pl.kernel(
        functools.partial(_spmm_body, nchunk),
        out_type=jax.ShapeDtypeStruct((n_rows, D), jnp.float32),
        mesh=plsc.VectorSubcoreMesh(core_axis_name="c", subcore_axis_name="s"),
        compiler_params=pltpu.CompilerParams(use_tc_tiling_on_sc=False,
                                             needs_layout_passes=False),
        scratch_types=[
            pltpu.VMEM((NBUF, 3, CHUNK), jnp.int32),
            pltpu.VMEM((NBUF, CHUNK, D), jnp.float32),
            pltpu.VMEM((NBUF, CHUNK), jnp.int32),
            pltpu.VMEM_SHARED((ACC_ROWS, D), jnp.float32),
        ] + [pltpu.SemaphoreType.DMA] * (3 * NBUF),
    )


def _sums_body(e0u, z1u, z2u, z3u, e0i, z1i, z2i, z3i,
               eu, su, ei, si):
    pu = e0u[...] + z1u[...] + z2u[...]
    su[...] = pu
    eu[...] = pu + z3u[...]
    pi = e0i[...] + z1i[...] + z2i[...]
    si[...] = pi
    ei[...] = pi + z3i[...]


def _kt_body(vtp, s_i, pu):
    pu[...] = jnp.dot(vtp[...], s_i[...], preferred_element_type=jnp.float32)


def _g_body(e0u, up, pu, e0i, vp, qi, gu, gi):
    gu[...] = e0u[...] + jnp.dot(up[...], pu[...],
                                 preferred_element_type=jnp.float32)
    gi[...] = e0i[...] + jnp.dot(vp[...], qi[...],
                                 preferred_element_type=jnp.float32)


def kernel(adj_indices, adj_values, E_u_0, E_i_0, u_mul_s, v_mul_s, ut, vt):
    n_users, _ = E_u_0.shape
    n_items, _ = E_i_0.shape
    nnz = adj_values.shape[0]
    row = adj_indices[0].astype(jnp.int32)
    col = adj_indices[1].astype(jnp.int32)
    val = adj_values.astype(jnp.float32)

    grp = NTILES * CHUNK * NBUF
    epad = -(-nnz // grp) * grp
    pad = epad - nnz
    if pad:
        row = jnp.concatenate([row, jnp.zeros((pad,), jnp.int32)])
        col = jnp.concatenate([col, jnp.zeros((pad,), jnp.int32)])
        val = jnp.concatenate([val, jnp.zeros((pad,), jnp.float32)])
    nchunk = epad // (NTILES * CHUNK)

    vbits = jax.lax.bitcast_convert_type(val, jnp.int32)
    pk_u = jnp.stack([row, col, vbits]).reshape(3, -1, CHUNK).transpose(1, 0, 2)
    pk_i = jnp.stack([col, row, vbits]).reshape(3, -1, CHUNK).transpose(1, 0, 2)

    spmm_u = _make_spmm(n_users, nchunk)
    spmm_i = _make_spmm(n_items, nchunk)

    Zu1 = spmm_u(pk_u, E_i_0)
    Zi1 = spmm_i(pk_i, E_u_0)
    Zu2 = spmm_u(pk_u, Zi1)
    Zi2 = spmm_i(pk_i, Zu1)
    Zu3 = spmm_u(pk_u, Zi2)
    Zi3 = spmm_i(pk_i, Zu2)

    nb = 50
    bu = n_users // nb
    bi = n_items // nb
    blk_u = pl.BlockSpec((bu, D), lambda i: (i, 0))
    blk_i = pl.BlockSpec((bi, D), lambda i: (i, 0))
    E_u, S_u, E_i, S_i = pl.pallas_call(
        _sums_body,
        grid=(nb,),
        in_specs=[blk_u] * 4 + [blk_i] * 4,
        out_specs=[blk_u, blk_u, blk_i, blk_i],
        out_shape=[jax.ShapeDtypeStruct((n_users, D), jnp.float32)] * 2
        + [jax.ShapeDtypeStruct((n_items, D), jnp.float32)] * 2,
    )(E_u_0, Zu1, Zu2, Zu3, E_i_0, Zi1, Zi2, Zi3)

    q = ut.shape[0]
    qp = 32
    vtp = jnp.pad(vt, ((0, qp - q), (0, 0)))
    utp = jnp.pad(ut, ((0, qp - q), (0, 0)))
    up = jnp.pad(u_mul_s, ((0, 0), (0, qp - q)))
    vp = jnp.pad(v_mul_s, ((0, 0), (0, qp - q)))

    P_u = pl.pallas_call(
        _kt_body,
        out_shape=jax.ShapeDtypeStruct((qp, D), jnp.float32),
    )(vtp, S_i)
    Q_i = pl.pallas_call(
        _kt_body,
        out_shape=jax.ShapeDtypeStruct((qp, D), jnp.float32),
    )(utp, S_u)

    G_u, G_i = pl.pallas_call(
        _g_body,
        grid=(nb,),
        in_specs=[
            blk_u,
            pl.BlockSpec((bu, qp), lambda i: (i, 0)),
            pl.BlockSpec((qp, D), lambda i: (0, 0)),
            blk_i,
            pl.BlockSpec((bi, qp), lambda i: (i, 0)),
            pl.BlockSpec((qp, D), lambda i: (0, 0)),
        ],
        out_specs=[blk_u, blk_i],
        out_shape=[jax.ShapeDtypeStruct((n_users, D), jnp.float32),
                   jax.ShapeDtypeStruct((n_items, D), jnp.float32)],
    )(E_u_0, up, P_u, E_i_0, vp, Q_i)

    return (E_u, E_i, G_u, G_i)


# speedup vs baseline: 1.3499x; 1.2336x over previous
import functools

import jax
import jax.numpy as jnp
from jax import lax
from jax.experimental import pallas as pl
from jax.experimental.pallas import tpu as pltpu
from jax.experimental.pallas import tpu_sc as plsc

D = 64
CHUNK = 128
NTILES = 16
NCORES = 2
HALF = 25000
ACC_ROWS = 196 * 128
OUT_PER_TILE = 1560
NBUF = 2
QPERM = sum(([k, 16 + k] for k in range(16)), [])
QPERM += sum(([32 + k, 48 + k] for k in range(16)), [])


def _spmm_body(nchunk, pk_hbm, table_hbm, out_hbm,
               pkb4, gbuf4, rows4, sidx4, acc_sh, *sems):
    c = lax.axis_index("c")
    s = lax.axis_index("s")
    si = sems[0:NBUF]
    sg = sems[NBUF:2 * NBUF]
    ss = sems[2 * NBUF:3 * NBUF]

    zero16 = jnp.zeros((16,), jnp.float32)

    def zrow(e, _):
        for j in range(D // 16):
            rows4[0, e, pl.ds(16 * j, 16)] = zero16
        return _

    lax.fori_loop(0, CHUNK, zrow, 0)
    nzc = ACC_ROWS // CHUNK
    for k in range(-(-nzc // NTILES)):
        zc = k * NTILES + s

        @pl.when(zc < nzc)
        def _():
            pltpu.sync_copy(rows4.at[0], acc_sh.at[pl.ds(zc * CHUNK, CHUNK)])
    plsc.subcore_barrier()

    base_cid = s * nchunk

    def start_idx(ci, b):
        pltpu.async_copy(pk_hbm.at[base_cid + ci], pkb4.at[b], si[b])

    def wait_idx(ci, b):
        pltpu.make_async_copy(pk_hbm.at[base_cid + ci], pkb4.at[b],
                              si[b]).wait()

    def start_gather(b):
        pltpu.async_copy(table_hbm.at[pkb4.at[b].at[1]], gbuf4.at[b], sg[b])

    def wait_gather(b):
        pltpu.make_async_copy(table_hbm.at[pkb4.at[b].at[1]], gbuf4.at[b],
                              sg[b]).wait()

    def start_scatter(b):
        pltpu.async_copy(rows4.at[b], acc_sh.at[sidx4.at[b]], ss[b], add=True)

    def wait_scatter(b):
        pltpu.make_async_copy(rows4.at[b], acc_sh.at[sidx4.at[b]],
                              ss[b]).wait()

    def compute(b):
        half_base = c * HALF
        trash = HALF + lax.iota(jnp.int32, 16)
        for j in range(CHUNK // 16):
            d16 = pkb4[b, 0, pl.ds(16 * j, 16)]
            loc = d16 - half_base
            ok = (loc >= 0) & (loc < HALF)
            sidx4[b, pl.ds(16 * j, 16)] = jnp.where(ok, loc, trash)
        himask = jnp.int32(-65536)
        for j in range(CHUNK // 16):
            vv = plsc.bitcast(pkb4[b, 2, pl.ds(16 * j, 16)], jnp.float32)
            for t in range(16):
                e = 16 * j + t
                v = vv[t]
                wlo = gbuf4[b, e, pl.ds(0, 16)]
                whi = gbuf4[b, e, pl.ds(16, 16)]
                rows4[b, e, pl.ds(0, 16)] = (
                    plsc.bitcast(wlo << 16, jnp.float32) * v)
                rows4[b, e, pl.ds(16, 16)] = (
                    plsc.bitcast(wlo & himask, jnp.float32) * v)
                rows4[b, e, pl.ds(32, 16)] = (
                    plsc.bitcast(whi << 16, jnp.float32) * v)
                rows4[b, e, pl.ds(48, 16)] = (
                    plsc.bitcast(whi & himask, jnp.float32) * v)

    for k in range(NBUF):
        start_idx(k, k)
    for k in range(NBUF - 1):
        wait_idx(k, k)
        start_gather(k)

    def ring_body(cg, carry):
        for b in range(NBUF):
            ci = cg * NBUF + b
            b1 = (b + NBUF - 1) % NBUF

            @pl.when((ci >= 1) & (ci + NBUF - 1 < nchunk))
            def _():
                wait_scatter(b1)

            @pl.when(ci + NBUF - 1 < nchunk)
            def _():
                wait_idx(ci + NBUF - 1, b1)
                start_gather(b1)

            wait_gather(b)
            compute(b)

            @pl.when(ci + NBUF < nchunk)
            def _():
                start_idx(ci + NBUF, b)

            start_scatter(b)
        return carry

    lax.fori_loop(0, nchunk // NBUF, ring_body, 0)
    for b in range(NBUF):
        wait_scatter(b)
    plsc.subcore_barrier()

    @pl.when(s < NTILES - 1)
    def _():
        pltpu.sync_copy(
            acc_sh.at[pl.ds(s * OUT_PER_TILE, OUT_PER_TILE)],
            out_hbm.at[pl.ds(c * HALF + s * OUT_PER_TILE, OUT_PER_TILE)])

    @pl.when(s == NTILES - 1)
    def _():
        base = (NTILES - 1) * OUT_PER_TILE
        last = HALF - base
        pltpu.sync_copy(
            acc_sh.at[pl.ds(base, last)],
            out_hbm.at[pl.ds(c * HALF + base, last)])


@functools.cache
def _make_spmm(n_rows, nchunk):
    return pl.kernel(
        functools.partial(_spmm_body, nchunk),
        out_type=jax.ShapeDtypeStruct((n_rows, D), jnp.float32),
        mesh=plsc.VectorSubcoreMesh(core_axis_name="c", subcore_axis_name="s"),
        compiler_params=pltpu.CompilerParams(use_tc_tiling_on_sc=False,
                                             needs_layout_passes=False),
        scratch_types=[
            pltpu.VMEM((NBUF, 3, CHUNK), jnp.int32),
            pltpu.VMEM((NBUF, CHUNK, D // 2), jnp.int32),
            pltpu.VMEM((NBUF, CHUNK, D), jnp.float32),
            pltpu.VMEM((NBUF, CHUNK), jnp.int32),
            pltpu.VMEM_SHARED((ACC_ROWS, D), jnp.float32),
        ] + [pltpu.SemaphoreType.DMA] * (3 * NBUF),
    )


def _sums_body(e0u, z1u, z2u, z3u, e0i, z1i, z2i, z3i,
               eu, su, ei, si):
    pu = e0u[...] + z1u[...] + z2u[...]
    su[...] = pu
    eu[...] = pu + z3u[...]
    pi = e0i[...] + z1i[...] + z2i[...]
    si[...] = pi
    ei[...] = pi + z3i[...]


def _kt_body(vtp, s_i, pu):
    pu[...] = jnp.dot(vtp[...], s_i[...], preferred_element_type=jnp.float32)


def _g_body(e0u, up, pu, e0i, vp, qi, gu, gi):
    gu[...] = e0u[...] + jnp.dot(up[...], pu[...],
                                 preferred_element_type=jnp.float32)
    gi[...] = e0i[...] + jnp.dot(vp[...], qi[...],
                                 preferred_element_type=jnp.float32)


def kernel(adj_indices, adj_values, E_u_0, E_i_0, u_mul_s, v_mul_s, ut, vt):
    n_users, _ = E_u_0.shape
    n_items, _ = E_i_0.shape
    nnz = adj_values.shape[0]
    row = adj_indices[0].astype(jnp.int32)
    col = adj_indices[1].astype(jnp.int32)
    val = adj_values.astype(jnp.float32)

    grp = NTILES * CHUNK * NBUF
    epad = -(-nnz // grp) * grp
    pad = epad - nnz
    if pad:
        row = jnp.concatenate([row, jnp.zeros((pad,), jnp.int32)])
        col = jnp.concatenate([col, jnp.zeros((pad,), jnp.int32)])
        val = jnp.concatenate([val, jnp.zeros((pad,), jnp.float32)])
    nchunk = epad // (NTILES * CHUNK)

    vbits = jax.lax.bitcast_convert_type(val, jnp.int32)
    pk_u = jnp.stack([row, col, vbits]).reshape(3, -1, CHUNK).transpose(1, 0, 2)
    pk_i = jnp.stack([col, row, vbits]).reshape(3, -1, CHUNK).transpose(1, 0, 2)

    spmm_u = _make_spmm(n_users, nchunk)
    spmm_i = _make_spmm(n_items, nchunk)

    qperm = jnp.asarray(QPERM, jnp.int32)

    def pack_table(t):
        tq = t[:, qperm].astype(jnp.bfloat16)
        return jax.lax.bitcast_convert_type(
            tq.reshape(t.shape[0], D // 2, 2), jnp.int32)

    Zu1 = spmm_u(pk_u, pack_table(E_i_0))
    Zi1 = spmm_i(pk_i, pack_table(E_u_0))
    Zu2 = spmm_u(pk_u, pack_table(Zi1))
    Zi2 = spmm_i(pk_i, pack_table(Zu1))
    Zu3 = spmm_u(pk_u, pack_table(Zi2))
    Zi3 = spmm_i(pk_i, pack_table(Zu2))

    nb = 50
    bu = n_users // nb
    bi = n_items // nb
    blk_u = pl.BlockSpec((bu, D), lambda i: (i, 0))
    blk_i = pl.BlockSpec((bi, D), lambda i: (i, 0))
    E_u, S_u, E_i, S_i = pl.pallas_call(
        _sums_body,
        grid=(nb,),
        in_specs=[blk_u] * 4 + [blk_i] * 4,
        out_specs=[blk_u, blk_u, blk_i, blk_i],
        out_shape=[jax.ShapeDtypeStruct((n_users, D), jnp.float32)] * 2
        + [jax.ShapeDtypeStruct((n_items, D), jnp.float32)] * 2,
    )(E_u_0, Zu1, Zu2, Zu3, E_i_0, Zi1, Zi2, Zi3)

    q = ut.shape[0]
    qp = 32
    vtp = jnp.pad(vt, ((0, qp - q), (0, 0)))
    utp = jnp.pad(ut, ((0, qp - q), (0, 0)))
    up = jnp.pad(u_mul_s, ((0, 0), (0, qp - q)))
    vp = jnp.pad(v_mul_s, ((0, 0), (0, qp - q)))

    P_u = pl.pallas_call(
        _kt_body,
        out_shape=jax.ShapeDtypeStruct((qp, D), jnp.float32),
    )(vtp, S_i)
    Q_i = pl.pallas_call(
        _kt_body,
        out_shape=jax.ShapeDtypeStruct((qp, D), jnp.float32),
    )(utp, S_u)

    G_u, G_i = pl.pallas_call(
        _g_body,
        grid=(nb,),
        in_specs=[
            blk_u,
            pl.BlockSpec((bu, qp), lambda i: (i, 0)),
            pl.BlockSpec((qp, D), lambda i: (0, 0)),
            blk_i,
            pl.BlockSpec((bi, qp), lambda i: (i, 0)),
            pl.BlockSpec((qp, D), lambda i: (0, 0)),
        ],
        out_specs=[blk_u, blk_i],
        out_shape=[jax.ShapeDtypeStruct((n_users, D), jnp.float32),
                   jax.ShapeDtypeStruct((n_items, D), jnp.float32)],
    )(E_u_0, up, P_u, E_i_0, vp, Q_i)

    return (E_u, E_i, G_u, G_i)


# speedup vs baseline: 1.5111x; 1.1194x over previous
import functools

import jax
import jax.numpy as jnp
from jax import lax
from jax.experimental import pallas as pl
from jax.experimental.pallas import tpu as pltpu
from jax.experimental.pallas import tpu_sc as plsc

D = 64
CHUNK = 128
NTILES = 16
NCORES = 2
HALF = 25000
ACC_ROWS = 196 * 128
OUT_PER_TILE = 1560
NBUF = 2
QPERM = sum(([k, 16 + k] for k in range(16)), [])
QPERM += sum(([32 + k, 48 + k] for k in range(16)), [])


def _spmm_body(nchunk, pk_hbm, table_hbm, out_hbm,
               pkb4, gbuf4, rows4, sidx4, acc_sh, *sems):
    c = lax.axis_index("c")
    s = lax.axis_index("s")
    si = sems[0:NBUF]
    sg = sems[NBUF:2 * NBUF]
    ss = sems[2 * NBUF:3 * NBUF]

    zero16 = jnp.zeros((16,), jnp.float32)

    def zrow(e, _):
        for j in range(D // 16):
            rows4[0, e, pl.ds(16 * j, 16)] = zero16
        return _

    lax.fori_loop(0, CHUNK, zrow, 0)
    nzc = ACC_ROWS // CHUNK
    for k in range(-(-nzc // NTILES)):
        zc = k * NTILES + s

        @pl.when(zc < nzc)
        def _():
            pltpu.sync_copy(rows4.at[0], acc_sh.at[pl.ds(zc * CHUNK, CHUNK)])
    plsc.subcore_barrier()

    base_cid = s * nchunk

    def start_idx(ci, b):
        pltpu.async_copy(pk_hbm.at[base_cid + ci], pkb4.at[b], si[b])

    def wait_idx(ci, b):
        pltpu.make_async_copy(pk_hbm.at[base_cid + ci], pkb4.at[b],
                              si[b]).wait()

    def start_gather(b):
        pltpu.async_copy(table_hbm.at[pkb4.at[b].at[1]], gbuf4.at[b], sg[b])

    def wait_gather(b):
        pltpu.make_async_copy(table_hbm.at[pkb4.at[b].at[1]], gbuf4.at[b],
                              sg[b]).wait()

    def start_scatter(b):
        pltpu.async_copy(rows4.at[b], acc_sh.at[sidx4.at[b]], ss[b], add=True)

    def wait_scatter(b):
        pltpu.make_async_copy(rows4.at[b], acc_sh.at[sidx4.at[b]],
                              ss[b]).wait()

    def compute(b):
        half_base = c * HALF
        trash = HALF + lax.iota(jnp.int32, 16)
        for j in range(CHUNK // 16):
            d16 = pkb4[b, 0, pl.ds(16 * j, 16)]
            loc = d16 - half_base
            ok = (loc >= 0) & (loc < HALF)
            sidx4[b, pl.ds(16 * j, 16)] = jnp.where(ok, loc, trash)
        himask = jnp.int32(-65536)
        for j in range(CHUNK // 16):
            vv = plsc.bitcast(pkb4[b, 2, pl.ds(16 * j, 16)], jnp.float32)
            for t in range(16):
                e = 16 * j + t
                v = vv[t]
                wlo = gbuf4[b, e, pl.ds(0, 16)]
                whi = gbuf4[b, e, pl.ds(16, 16)]
                rows4[b, e, pl.ds(0, 16)] = (
                    plsc.bitcast(wlo << 16, jnp.float32) * v)
                rows4[b, e, pl.ds(16, 16)] = (
                    plsc.bitcast(wlo & himask, jnp.float32) * v)
                rows4[b, e, pl.ds(32, 16)] = (
                    plsc.bitcast(whi << 16, jnp.float32) * v)
                rows4[b, e, pl.ds(48, 16)] = (
                    plsc.bitcast(whi & himask, jnp.float32) * v)

    for k in range(NBUF):
        start_idx(k, k)
    for k in range(NBUF - 1):
        wait_idx(k, k)
        start_gather(k)

    def ring_body(cg, carry):
        for b in range(NBUF):
            ci = cg * NBUF + b
            b1 = (b + NBUF - 1) % NBUF

            @pl.when((ci >= 1) & (ci + NBUF - 1 < nchunk))
            def _():
                wait_scatter(b1)

            @pl.when(ci + NBUF - 1 < nchunk)
            def _():
                wait_idx(ci + NBUF - 1, b1)
                start_gather(b1)

            wait_gather(b)
            compute(b)

            @pl.when(ci + NBUF < nchunk)
            def _():
                start_idx(ci + NBUF, b)

            start_scatter(b)
        return carry

    lax.fori_loop(0, nchunk // NBUF, ring_body, 0)
    for b in range(NBUF):
        wait_scatter(b)
    plsc.subcore_barrier()

    @pl.when(s < NTILES - 1)
    def _():
        pltpu.sync_copy(
            acc_sh.at[pl.ds(s * OUT_PER_TILE, OUT_PER_TILE)],
            out_hbm.at[pl.ds(c * HALF + s * OUT_PER_TILE, OUT_PER_TILE)])

    @pl.when(s == NTILES - 1)
    def _():
        base = (NTILES - 1) * OUT_PER_TILE
        last = HALF - base
        pltpu.sync_copy(
            acc_sh.at[pl.ds(base, last)],
            out_hbm.at[pl.ds(c * HALF + base, last)])


@functools.cache
def _make_spmm(n_rows, nchunk):
    return pl.kernel(
        functools.partial(_spmm_body, nchunk),
        out_type=jax.ShapeDtypeStruct((n_rows, D), jnp.float32),
        mesh=plsc.VectorSubcoreMesh(core_axis_name="c", subcore_axis_name="s"),
        compiler_params=pltpu.CompilerParams(use_tc_tiling_on_sc=False,
                                             needs_layout_passes=False),
        scratch_types=[
            pltpu.VMEM((NBUF, 3, CHUNK), jnp.int32),
            pltpu.VMEM((NBUF, CHUNK, D // 2), jnp.int32),
            pltpu.VMEM((NBUF, CHUNK, D), jnp.float32),
            pltpu.VMEM((NBUF, CHUNK), jnp.int32),
            pltpu.VMEM_SHARED((ACC_ROWS, D), jnp.float32),
        ] + [pltpu.SemaphoreType.DMA] * (3 * NBUF),
    )


def _sums_body(e0u, z1u, z2u, z3u, e0i, z1i, z2i, z3i,
               eu, su, ei, si):
    pu = e0u[...] + z1u[...] + z2u[...]
    su[...] = pu
    eu[...] = pu + z3u[...]
    pi = e0i[...] + z1i[...] + z2i[...]
    si[...] = pi
    ei[...] = pi + z3i[...]


def _kt_body(vtp, s_i, pu):
    pu[...] = jnp.dot(vtp[...], s_i[...], preferred_element_type=jnp.float32)


def _g_body(e0u, up, pu, e0i, vp, qi, gu, gi):
    gu[...] = e0u[...] + jnp.dot(up[...], pu[...],
                                 preferred_element_type=jnp.float32)
    gi[...] = e0i[...] + jnp.dot(vp[...], qi[...],
                                 preferred_element_type=jnp.float32)


def kernel(adj_indices, adj_values, E_u_0, E_i_0, u_mul_s, v_mul_s, ut, vt):
    n_users, _ = E_u_0.shape
    n_items, _ = E_i_0.shape
    nnz = adj_values.shape[0]
    row = adj_indices[0].astype(jnp.int32)
    col = adj_indices[1].astype(jnp.int32)
    val = adj_values.astype(jnp.float32)

    grp = NTILES * CHUNK * NBUF
    epad = -(-nnz // grp) * grp
    pad = epad - nnz
    if pad:
        row = jnp.concatenate([row, jnp.zeros((pad,), jnp.int32)])
        col = jnp.concatenate([col, jnp.zeros((pad,), jnp.int32)])
        val = jnp.concatenate([val, jnp.zeros((pad,), jnp.float32)])
    nchunk = epad // (NTILES * CHUNK)

    vbits = jax.lax.bitcast_convert_type(val, jnp.int32)
    pk_u = jnp.stack([row, col, vbits]).reshape(3, -1, CHUNK).transpose(1, 0, 2)
    pk_i = jnp.stack([col, row, vbits]).reshape(3, -1, CHUNK).transpose(1, 0, 2)

    spmm_u = _make_spmm(n_users, nchunk)
    spmm_i = _make_spmm(n_items, nchunk)

    def pack_table(t):
        n = t.shape[0]
        tq = (t.reshape(n, 2, 2, 16).transpose(0, 1, 3, 2)
              .astype(jnp.bfloat16))
        return jax.lax.bitcast_convert_type(tq, jnp.int32).reshape(n, D // 2)

    Zu1 = spmm_u(pk_u, pack_table(E_i_0))
    Zi1 = spmm_i(pk_i, pack_table(E_u_0))
    Zu2 = spmm_u(pk_u, pack_table(Zi1))
    Zi2 = spmm_i(pk_i, pack_table(Zu1))
    Zu3 = spmm_u(pk_u, pack_table(Zi2))
    Zi3 = spmm_i(pk_i, pack_table(Zu2))

    nb = 50
    bu = n_users // nb
    bi = n_items // nb
    blk_u = pl.BlockSpec((bu, D), lambda i: (i, 0))
    blk_i = pl.BlockSpec((bi, D), lambda i: (i, 0))
    E_u, S_u, E_i, S_i = pl.pallas_call(
        _sums_body,
        grid=(nb,),
        in_specs=[blk_u] * 4 + [blk_i] * 4,
        out_specs=[blk_u, blk_u, blk_i, blk_i],
        out_shape=[jax.ShapeDtypeStruct((n_users, D), jnp.float32)] * 2
        + [jax.ShapeDtypeStruct((n_items, D), jnp.float32)] * 2,
    )(E_u_0, Zu1, Zu2, Zu3, E_i_0, Zi1, Zi2, Zi3)

    q = ut.shape[0]
    qp = 32
    vtp = jnp.pad(vt, ((0, qp - q), (0, 0)))
    utp = jnp.pad(ut, ((0, qp - q), (0, 0)))
    up = jnp.pad(u_mul_s, ((0, 0), (0, qp - q)))
    vp = jnp.pad(v_mul_s, ((0, 0), (0, qp - q)))

    P_u = pl.pallas_call(
        _kt_body,
        out_shape=jax.ShapeDtypeStruct((qp, D), jnp.float32),
    )(vtp, S_i)
    Q_i = pl.pallas_call(
        _kt_body,
        out_shape=jax.ShapeDtypeStruct((qp, D), jnp.float32),
    )(utp, S_u)

    G_u, G_i = pl.pallas_call(
        _g_body,
        grid=(nb,),
        in_specs=[
            blk_u,
            pl.BlockSpec((bu, qp), lambda i: (i, 0)),
            pl.BlockSpec((qp, D), lambda i: (0, 0)),
            blk_i,
            pl.BlockSpec((bi, qp), lambda i: (i, 0)),
            pl.BlockSpec((qp, D), lambda i: (0, 0)),
        ],
        out_specs=[blk_u, blk_i],
        out_shape=[jax.ShapeDtypeStruct((n_users, D), jnp.float32),
                   jax.ShapeDtypeStruct((n_items, D), jnp.float32)],
    )(E_u_0, up, P_u, E_i_0, vp, Q_i)

    return (E_u, E_i, G_u, G_i)


# speedup vs baseline: 1.5159x; 1.0032x over previous
import functools

import jax
import jax.numpy as jnp
from jax import lax
from jax.experimental import pallas as pl
from jax.experimental.pallas import tpu as pltpu
from jax.experimental.pallas import tpu_sc as plsc

D = 64
CHUNK = 128
NTILES = 16
NCORES = 2
HALF = 25000
ACC_ROWS = 196 * 128
OUT_PER_TILE = 1560
NBUF = 2
QPERM = sum(([k, 16 + k] for k in range(16)), [])
QPERM += sum(([32 + k, 48 + k] for k in range(16)), [])


def _spmm_body(nchunk, pk_hbm, table_hbm, out_hbm,
               pkb4, gbuf4, rows4, sidx4, acc_sh, *sems):
    c = lax.axis_index("c")
    s = lax.axis_index("s")
    si = sems[0:NBUF]
    sg = sems[NBUF:2 * NBUF]
    ss = sems[2 * NBUF:3 * NBUF]

    zero16 = jnp.zeros((16,), jnp.float32)

    def zrow(e, _):
        for j in range(D // 16):
            rows4[0, e, pl.ds(16 * j, 16)] = zero16
        return _

    base_cid = s * nchunk

    def start_idx(ci, b):
        pltpu.async_copy(pk_hbm.at[base_cid + ci], pkb4.at[b], si[b])

    def wait_idx(ci, b):
        pltpu.make_async_copy(pk_hbm.at[base_cid + ci], pkb4.at[b],
                              si[b]).wait()

    def start_gather(b):
        pltpu.async_copy(table_hbm.at[pkb4.at[b].at[1]], gbuf4.at[b], sg[b])

    def wait_gather(b):
        pltpu.make_async_copy(table_hbm.at[pkb4.at[b].at[1]], gbuf4.at[b],
                              sg[b]).wait()

    def start_scatter(b):
        pltpu.async_copy(rows4.at[b], acc_sh.at[sidx4.at[b]], ss[b], add=True)

    def wait_scatter(b):
        pltpu.make_async_copy(rows4.at[b], acc_sh.at[sidx4.at[b]],
                              ss[b]).wait()

    def compute(b):
        half_base = c * HALF
        trash = HALF + lax.iota(jnp.int32, 16)
        for j in range(CHUNK // 16):
            d16 = pkb4[b, 0, pl.ds(16 * j, 16)]
            loc = d16 - half_base
            ok = (loc >= 0) & (loc < HALF)
            sidx4[b, pl.ds(16 * j, 16)] = jnp.where(ok, loc, trash)
        himask = jnp.int32(-65536)
        for j in range(CHUNK // 16):
            vv = plsc.bitcast(pkb4[b, 2, pl.ds(16 * j, 16)], jnp.float32)
            for t in range(16):
                e = 16 * j + t
                v = vv[t]
                wlo = gbuf4[b, e, pl.ds(0, 16)]
                whi = gbuf4[b, e, pl.ds(16, 16)]
                rows4[b, e, pl.ds(0, 16)] = (
                    plsc.bitcast(wlo << 16, jnp.float32) * v)
                rows4[b, e, pl.ds(16, 16)] = (
                    plsc.bitcast(wlo & himask, jnp.float32) * v)
                rows4[b, e, pl.ds(32, 16)] = (
                    plsc.bitcast(whi << 16, jnp.float32) * v)
                rows4[b, e, pl.ds(48, 16)] = (
                    plsc.bitcast(whi & himask, jnp.float32) * v)

    for k in range(NBUF):
        start_idx(k, k)
    for k in range(NBUF - 1):
        wait_idx(k, k)
        start_gather(k)

    lax.fori_loop(0, CHUNK, zrow, 0)
    nzc = ACC_ROWS // CHUNK
    for k in range(-(-nzc // NTILES)):
        zc = k * NTILES + s

        @pl.when(zc < nzc)
        def _():
            pltpu.sync_copy(rows4.at[0], acc_sh.at[pl.ds(zc * CHUNK, CHUNK)])
    plsc.subcore_barrier()

    def ring_body(cg, carry):
        for b in range(NBUF):
            ci = cg * NBUF + b
            b1 = (b + NBUF - 1) % NBUF

            @pl.when((ci >= 1) & (ci + NBUF - 1 < nchunk))
            def _():
                wait_scatter(b1)

            @pl.when(ci + NBUF - 1 < nchunk)
            def _():
                wait_idx(ci + NBUF - 1, b1)
                start_gather(b1)

            wait_gather(b)
            compute(b)

            @pl.when(ci + NBUF < nchunk)
            def _():
                start_idx(ci + NBUF, b)

            start_scatter(b)
        return carry

    lax.fori_loop(0, nchunk // NBUF, ring_body, 0)
    for b in range(NBUF):
        wait_scatter(b)
    plsc.subcore_barrier()

    @pl.when(s < NTILES - 1)
    def _():
        pltpu.sync_copy(
            acc_sh.at[pl.ds(s * OUT_PER_TILE, OUT_PER_TILE)],
            out_hbm.at[pl.ds(c * HALF + s * OUT_PER_TILE, OUT_PER_TILE)])

    @pl.when(s == NTILES - 1)
    def _():
        base = (NTILES - 1) * OUT_PER_TILE
        last = HALF - base
        pltpu.sync_copy(
            acc_sh.at[pl.ds(base, last)],
            out_hbm.at[pl.ds(c * HALF + base, last)])


@functools.cache
def _make_spmm(n_rows, nchunk):
    return pl.kernel(
        functools.partial(_spmm_body, nchunk),
        out_type=jax.ShapeDtypeStruct((n_rows, D), jnp.float32),
        mesh=plsc.VectorSubcoreMesh(core_axis_name="c", subcore_axis_name="s"),
        compiler_params=pltpu.CompilerParams(use_tc_tiling_on_sc=False,
                                             needs_layout_passes=False),
        scratch_types=[
            pltpu.VMEM((NBUF, 3, CHUNK), jnp.int32),
            pltpu.VMEM((NBUF, CHUNK, D // 2), jnp.int32),
            pltpu.VMEM((NBUF, CHUNK, D), jnp.float32),
            pltpu.VMEM((NBUF, CHUNK), jnp.int32),
            pltpu.VMEM_SHARED((ACC_ROWS, D), jnp.float32),
        ] + [pltpu.SemaphoreType.DMA] * (3 * NBUF),
    )


def _sums_body(e0u, z1u, z2u, z3u, e0i, z1i, z2i, z3i,
               eu, su, ei, si):
    pu = e0u[...] + z1u[...] + z2u[...]
    su[...] = pu
    eu[...] = pu + z3u[...]
    pi = e0i[...] + z1i[...] + z2i[...]
    si[...] = pi
    ei[...] = pi + z3i[...]


def _kt_body(vtp, s_i, pu):
    pu[...] = jnp.dot(vtp[...], s_i[...], preferred_element_type=jnp.float32)


def _g_body(e0u, up, pu, e0i, vp, qi, gu, gi):
    gu[...] = e0u[...] + jnp.dot(up[...], pu[...],
                                 preferred_element_type=jnp.float32)
    gi[...] = e0i[...] + jnp.dot(vp[...], qi[...],
                                 preferred_element_type=jnp.float32)


def kernel(adj_indices, adj_values, E_u_0, E_i_0, u_mul_s, v_mul_s, ut, vt):
    n_users, _ = E_u_0.shape
    n_items, _ = E_i_0.shape
    nnz = adj_values.shape[0]
    row = adj_indices[0].astype(jnp.int32)
    col = adj_indices[1].astype(jnp.int32)
    val = adj_values.astype(jnp.float32)

    grp = NTILES * CHUNK * NBUF
    epad = -(-nnz // grp) * grp
    pad = epad - nnz
    if pad:
        row = jnp.concatenate([row, jnp.zeros((pad,), jnp.int32)])
        col = jnp.concatenate([col, jnp.zeros((pad,), jnp.int32)])
        val = jnp.concatenate([val, jnp.zeros((pad,), jnp.float32)])
    nchunk = epad // (NTILES * CHUNK)

    vbits = jax.lax.bitcast_convert_type(val, jnp.int32)
    pk_u = jnp.stack([row, col, vbits]).reshape(3, -1, CHUNK).transpose(1, 0, 2)
    pk_i = jnp.stack([col, row, vbits]).reshape(3, -1, CHUNK).transpose(1, 0, 2)

    spmm_u = _make_spmm(n_users, nchunk)
    spmm_i = _make_spmm(n_items, nchunk)

    def pack_table(t):
        n = t.shape[0]
        tq = (t.reshape(n, 2, 2, 16).transpose(0, 1, 3, 2)
              .astype(jnp.bfloat16))
        return jax.lax.bitcast_convert_type(tq, jnp.int32).reshape(n, D // 2)

    Zu1 = spmm_u(pk_u, pack_table(E_i_0))
    Zi1 = spmm_i(pk_i, pack_table(E_u_0))
    Zu2 = spmm_u(pk_u, pack_table(Zi1))
    Zi2 = spmm_i(pk_i, pack_table(Zu1))
    Zu3 = spmm_u(pk_u, pack_table(Zi2))
    Zi3 = spmm_i(pk_i, pack_table(Zu2))

    nb = 50
    bu = n_users // nb
    bi = n_items // nb
    blk_u = pl.BlockSpec((bu, D), lambda i: (i, 0))
    blk_i = pl.BlockSpec((bi, D), lambda i: (i, 0))
    E_u, S_u, E_i, S_i = pl.pallas_call(
        _sums_body,
        grid=(nb,),
        in_specs=[blk_u] * 4 + [blk_i] * 4,
        out_specs=[blk_u, blk_u, blk_i, blk_i],
        out_shape=[jax.ShapeDtypeStruct((n_users, D), jnp.float32)] * 2
        + [jax.ShapeDtypeStruct((n_items, D), jnp.float32)] * 2,
    )(E_u_0, Zu1, Zu2, Zu3, E_i_0, Zi1, Zi2, Zi3)

    q = ut.shape[0]
    qp = 32
    vtp = jnp.pad(vt, ((0, qp - q), (0, 0)))
    utp = jnp.pad(ut, ((0, qp - q), (0, 0)))
    up = jnp.pad(u_mul_s, ((0, 0), (0, qp - q)))
    vp = jnp.pad(v_mul_s, ((0, 0), (0, qp - q)))

    P_u = pl.pallas_call(
        _kt_body,
        out_shape=jax.ShapeDtypeStruct((qp, D), jnp.float32),
    )(vtp, S_i)
    Q_i = pl.pallas_call(
        _kt_body,
        out_shape=jax.ShapeDtypeStruct((qp, D), jnp.float32),
    )(utp, S_u)

    G_u, G_i = pl.pallas_call(
        _g_body,
        grid=(nb,),
        in_specs=[
            blk_u,
            pl.BlockSpec((bu, qp), lambda i: (i, 0)),
            pl.BlockSpec((qp, D), lambda i: (0, 0)),
            blk_i,
            pl.BlockSpec((bi, qp), lambda i: (i, 0)),
            pl.BlockSpec((qp, D), lambda i: (0, 0)),
        ],
        out_specs=[blk_u, blk_i],
        out_shape=[jax.ShapeDtypeStruct((n_users, D), jnp.float32),
                   jax.ShapeDtypeStruct((n_items, D), jnp.float32)],
    )(E_u_0, up, P_u, E_i_0, vp, Q_i)

    return (E_u, E_i, G_u, G_i)
